# baseline (speedup 1.0000x reference)
import jax
import jax.numpy as jnp
from jax import lax
from jax.experimental import pallas as pl
from jax.experimental.pallas import tpu as pltpu
from jax.experimental.pallas import tpu_sc as plsc

N = 10000
E = 320000
D = 128
DH = D // 2
NP = 10240
ROWS_PER_SUB = NP // 16
CHUNK = 640
NGRP = CHUNK // 128
NCHUNKS = E // CHUNK
NSUB = 16
NSLOT = ((-(-NCHUNKS // NSUB) + 1) // 2) * 2


def _node_pre_body(x_ref, wt_ref, am_ref, h_ref, asd_ref):
    h = jnp.dot(x_ref[...], wt_ref[...], preferred_element_type=jnp.float32)
    h_ref[...] = h
    asd_ref[...] = jnp.dot(h, am_ref[...], preferred_element_type=jnp.float32)


def _node_pre(x, wt, attm):
    blk = 1000
    return pl.pallas_call(
        _node_pre_body,
        grid=(N // blk,),
        in_specs=[
            pl.BlockSpec((blk, D), lambda i: (i, 0)),
            pl.BlockSpec((D, D), lambda i: (0, 0)),
            pl.BlockSpec((D, D), lambda i: (0, 0)),
        ],
        out_specs=[
            pl.BlockSpec((blk, D), lambda i: (i, 0)),
            pl.BlockSpec((blk, D), lambda i: (i, 0)),
        ],
        out_shape=[
            jax.ShapeDtypeStruct((N, D), jnp.float32),
            jax.ShapeDtypeStruct((N, D), jnp.float32),
        ],
    )(x, wt, attm)


def _edge_pre_body(ea_ref, w1_ref, a1_ref, w2_ref, a2_ref, e1_ref, e2_ref):
    ea = ea_ref[...]
    z1 = jnp.maximum(
        jnp.dot(ea, w1_ref[...], preferred_element_type=jnp.float32), 0.0)
    e1_ref[...] = jnp.dot(z1, a1_ref[...], preferred_element_type=jnp.float32)
    z2 = jnp.maximum(
        jnp.dot(ea, w2_ref[...], preferred_element_type=jnp.float32), 0.0)
    e2_ref[...] = jnp.dot(z2, a2_ref[...], preferred_element_type=jnp.float32)


def _edge_pre(edge_attr, wet1, ap1, wet2, ap2):
    blk = 2560
    de = edge_attr.shape[1]
    return pl.pallas_call(
        _edge_pre_body,
        grid=(E // blk,),
        in_specs=[
            pl.BlockSpec((blk, de), lambda i: (i, 0)),
            pl.BlockSpec((de, D), lambda i: (0, 0)),
            pl.BlockSpec((D, 8), lambda i: (0, 0)),
            pl.BlockSpec((de, D), lambda i: (0, 0)),
            pl.BlockSpec((D, 8), lambda i: (0, 0)),
        ],
        out_specs=[
            pl.BlockSpec((blk, 8), lambda i: (i, 0)),
            pl.BlockSpec((blk, 8), lambda i: (i, 0)),
        ],
        out_shape=[
            jax.ShapeDtypeStruct((E, 8), jnp.float32),
            jax.ShapeDtypeStruct((E, 8), jnp.float32),
        ],
    )(edge_attr, wet1, ap1, wet2, ap2)


def _merge_rows(acc_ref, s_ref):
    agg = jnp.concatenate([acc_ref[0], acc_ref[1]], axis=-1)
    den = s_ref[...] + 1e-16
    return agg / den


def _combine_body(acc_ref, s_ref, wt_ref, am_ref, h_ref, asd_ref):
    o = _merge_rows(acc_ref, s_ref)
    o = jnp.where(o > 0.0, o, jnp.exp(o) - 1.0)
    h = jnp.dot(o, wt_ref[...], preferred_element_type=jnp.float32)
    h_ref[...] = h
    asd_ref[...] = jnp.dot(h, am_ref[...], preferred_element_type=jnp.float32)


def _combine(acc, s, wt, attm):
    blk = 1000
    return pl.pallas_call(
        _combine_body,
        grid=(N // blk,),
        in_specs=[
            pl.BlockSpec((2, blk, DH), lambda i: (0, i, 0)),
            pl.BlockSpec((blk, 1), lambda i: (i, 0)),
            pl.BlockSpec((D, D), lambda i: (0, 0)),
            pl.BlockSpec((D, D), lambda i: (0, 0)),
        ],
        out_specs=[
            pl.BlockSpec((blk, D), lambda i: (i, 0)),
            pl.BlockSpec((blk, D), lambda i: (i, 0)),
        ],
        out_shape=[
            jax.ShapeDtypeStruct((N, D), jnp.float32),
            jax.ShapeDtypeStruct((N, D), jnp.float32),
        ],
    )(acc, s, wt, attm)


def _post_body(acc_ref, s_ref, out_ref):
    out_ref[...] = _merge_rows(acc_ref, s_ref)


def _post(acc, s):
    blk = 1000
    return pl.pallas_call(
        _post_body,
        grid=(N // blk,),
        in_specs=[
            pl.BlockSpec((2, blk, DH), lambda i: (0, i, 0)),
            pl.BlockSpec((blk, 1), lambda i: (i, 0)),
        ],
        out_specs=pl.BlockSpec((blk, D), lambda i: (i, 0)),
        out_shape=jax.ShapeDtypeStruct((N, D), jnp.float32),
    )(acc, s)


def _sc_gat_body(hs_hbm, as_hbm, ad_hbm, ep_hbm, src_hbm, dst_hbm,
                 acc_out, s_out,
                 as_v, ad_v, src_v, dst_v, ep_v, w_v, rows_v,
                 acc_sh, s_sh, sem_g, sem_i0, sem_i1, sem_s):
    c = lax.axis_index("c")
    sid = lax.axis_index("s")
    sem_i = (sem_i0, sem_i1)

    pltpu.sync_copy(as_hbm, as_v)
    pltpu.sync_copy(ad_hbm, ad_v)

    zeros16 = jnp.zeros((16,), jnp.float32)

    def _zero_row(i, _):
        for k in range(DH // 16):
            rows_v[i, pl.ds(k * 16, 16)] = zeros16
        return 0

    lax.fori_loop(0, 128, _zero_row, 0)
    for i in range(8):
        w_v[pl.ds(i * 16, 16)] = zeros16

    base_n = sid * ROWS_PER_SUB
    for m in range(ROWS_PER_SUB // 128):
        pltpu.sync_copy(rows_v.at[pl.ds(0, 128)],
                        acc_sh.at[pl.ds(base_n + m * 128, 128)])
        pltpu.sync_copy(w_v.at[pl.ds(0, 128)],
                        s_sh.at[pl.ds(base_n + m * 128, 128)])
    plsc.subcore_barrier()

    def _cid(i):
        t = sid + i * NSUB
        return jnp.minimum(t, NCHUNKS - 1), t < NCHUNKS

    def _fire_idx(i, st):
        cid, _ = _cid(i)
        base = cid * CHUNK
        for j in range(NGRP):
            pltpu.async_copy(src_hbm.at[pl.ds(base + j * 128, 128)],
                             src_v.at[st, j], sem_i[st])
            pltpu.async_copy(dst_hbm.at[pl.ds(base + j * 128, 128)],
                             dst_v.at[st, j], sem_i[st])
            pltpu.async_copy(ep_hbm.at[pl.ds(base + j * 128, 128)],
                             ep_v.at[st, j], sem_i[st])

    def _wait_idx(st):
        dummy = src_hbm.at[pl.ds(0, 128)]
        for j in range(NGRP):
            pltpu.make_async_copy(dummy, src_v.at[st, j], sem_i[st]).wait()
            pltpu.make_async_copy(dummy, dst_v.at[st, j], sem_i[st]).wait()
            pltpu.make_async_copy(
                ep_hbm.at[pl.ds(0, 128)], ep_v.at[st, j], sem_i[st]).wait()

    def _compute_w(i, st):
        _, live = _cid(i)
        fac = jnp.where(live, 1.0, 0.0)
        for j in range(NGRP):
            for t in range(8):
                o = t * 16
                s16 = src_v[st, j, pl.ds(o, 16)]
                d16 = dst_v[st, j, pl.ds(o, 16)]
                e = (plsc.load_gather(as_v, [s16])
                     + plsc.load_gather(ad_v, [d16])
                     + ep_v[st, j, pl.ds(o, 16)])
                e = jnp.where(e >= 0.0, e, 0.2 * e)
                w_v[pl.ds(j * 128 + o, 16)] = jnp.exp(e) * fac

    def _scale():
        def _body(t, _):
            w16 = w_v[pl.ds(t * 16, 16)]
            r0 = t * 16
            for l in range(16):
                wi = w16[l]
                for k in range(DH // 16):
                    rows_v[r0 + l, pl.ds(k * 16, 16)] = (
                        rows_v[r0 + l, pl.ds(k * 16, 16)] * wi)
            return 0
        lax.fori_loop(0, CHUNK // 16, _body, 0)

    def _slot(i, st):
        _wait_idx(st)
        _fire_idx(i + 1, 1 - st)
        gcps = [pltpu.async_copy(hs_hbm.at[c].at[src_v.at[st, j]],
                                 rows_v.at[pl.ds(j * 128, 128)], sem_g)
                for j in range(NGRP)]
        _compute_w(i, st)
        for cp in gcps:
            cp.wait()
        _scale()
        scps = [pltpu.async_copy(rows_v.at[pl.ds(j * 128, 128)],
                                 acc_sh.at[dst_v.at[st, j]], sem_s,
                                 add=True)
                for j in range(NGRP)]
        for cp in scps:
            cp.wait()

        @pl.when(c == 0)
        def _():
            wcps = [pltpu.async_copy(w_v.at[pl.ds(j * 128, 128)],
                                     s_sh.at[dst_v.at[st, j]], sem_s,
                                     add=True)
                    for j in range(NGRP)]
            for cp in wcps:
                cp.wait()

    _fire_idx(jnp.int32(0), 0)

    def _bdy(k, _):
        _slot(k * 2, 0)
        _slot(k * 2 + 1, 1)
        return 0

    lax.fori_loop(0, NSLOT // 2, _bdy, 0)
    _wait_idx(0)
    plsc.subcore_barrier()

    for m in range(ROWS_PER_SUB // 128):
        r0 = base_n + m * 128
        pltpu.sync_copy(acc_sh.at[pl.ds(r0, 128)],
                        acc_out.at[c, pl.ds(r0, 128)])

    @pl.when(c == 0)
    def _():
        for m in range(ROWS_PER_SUB // 128):
            r0 = base_n + m * 128
            pltpu.sync_copy(s_sh.at[pl.ds(r0, 128)],
                            s_out.at[pl.ds(r0, 128)])


def _gat_layer(hs, a_s, a_d, ep, src, dst):
    mesh = plsc.VectorSubcoreMesh(core_axis_name="c", subcore_axis_name="s",
                                  num_cores=2, num_subcores=NSUB)
    f = pl.kernel(
        _sc_gat_body,
        out_type=[
            jax.ShapeDtypeStruct((2, NP, DH), jnp.float32),
            jax.ShapeDtypeStruct((NP,), jnp.float32),
        ],
        mesh=mesh,
        compiler_params=pltpu.CompilerParams(needs_layout_passes=False,
                                             use_tc_tiling_on_sc=False),
        scratch_types=[
            pltpu.VMEM((N,), jnp.float32),
            pltpu.VMEM((N,), jnp.float32),
            pltpu.VMEM((2, NGRP, 128), jnp.int32),
            pltpu.VMEM((2, NGRP, 128), jnp.int32),
            pltpu.VMEM((2, NGRP, 128), jnp.float32),
            pltpu.VMEM((CHUNK,), jnp.float32),
            pltpu.VMEM((CHUNK, DH), jnp.float32),
            pltpu.VMEM_SHARED((NP, DH), jnp.float32),
            pltpu.VMEM_SHARED((NP,), jnp.float32),
            pltpu.SemaphoreType.DMA,
            pltpu.SemaphoreType.DMA,
            pltpu.SemaphoreType.DMA,
            pltpu.SemaphoreType.DMA,
        ],
    )
    return f(hs, a_s, a_d, ep, src, dst)


def _att_mats(att):
    a = att[0]
    attm = jnp.zeros((D, D), jnp.float32)
    attm = attm.at[:, 0].set(a[:D]).at[:, 1].set(a[D:2 * D])
    attp = jnp.zeros((D, 8), jnp.float32).at[:, 0].set(a[2 * D:])
    return attm, attp


def _split_cols(h):
    return jnp.stack([h[:, :DH], h[:, DH:]])


def kernel(x, edge_index, edge_attr, W1, We1, att1, W2, We2, att2):
    src = edge_index[0]
    dst = edge_index[1]
    attm1, attp1 = _att_mats(att1)
    attm2, attp2 = _att_mats(att2)

    h1, asd1 = _node_pre(x, W1.T, attm1)
    ep1_8, ep2_8 = _edge_pre(edge_attr, We1.T, attp1, We2.T, attp2)
    ep1 = ep1_8[:, 0]
    ep2 = ep2_8[:, 0]

    acc1, s1 = _gat_layer(_split_cols(h1), asd1[:, 0], asd1[:, 1],
                          ep1, src, dst)
    h2, asd2 = _combine(acc1, s1.reshape(NP, 1), W2.T, attm2)
    acc2, s2 = _gat_layer(_split_cols(h2), asd2[:, 0], asd2[:, 1],
                          ep2, src, dst)
    return _post(acc2, s2.reshape(NP, 1))


# speedup vs baseline: 1.0008x; 1.0008x over previous
import jax
import jax.numpy as jnp
from jax import lax
from jax.experimental import pallas as pl
from jax.experimental.pallas import tpu as pltpu
from jax.experimental.pallas import tpu_sc as plsc

N = 10000
E = 320000
D = 128
DH = D // 2
NP = 10240
ROWS_PER_SUB = NP // 16
CHUNK = 640
NGRP = CHUNK // 128
NCHUNKS = E // CHUNK
NSUB = 16
NSLOT = ((-(-NCHUNKS // NSUB) + 1) // 2) * 2


def _node_pre_body(x_ref, wt_ref, am_ref, h_ref, asd_ref):
    h = jnp.dot(x_ref[...], wt_ref[...], preferred_element_type=jnp.float32)
    h_ref[...] = h
    asd_ref[...] = jnp.dot(h, am_ref[...], preferred_element_type=jnp.float32)


def _node_pre(x, wt, attm):
    blk = 1000
    return pl.pallas_call(
        _node_pre_body,
        grid=(N // blk,),
        in_specs=[
            pl.BlockSpec((blk, D), lambda i: (i, 0)),
            pl.BlockSpec((D, D), lambda i: (0, 0)),
            pl.BlockSpec((D, D), lambda i: (0, 0)),
        ],
        out_specs=[
            pl.BlockSpec((blk, D), lambda i: (i, 0)),
            pl.BlockSpec((blk, D), lambda i: (i, 0)),
        ],
        out_shape=[
            jax.ShapeDtypeStruct((N, D), jnp.float32),
            jax.ShapeDtypeStruct((N, D), jnp.float32),
        ],
    )(x, wt, attm)


def _edge_pre_body(ea_ref, w1_ref, a1_ref, w2_ref, a2_ref, e1_ref, e2_ref):
    ea = ea_ref[...]
    z1 = jnp.maximum(
        jnp.dot(ea, w1_ref[...], preferred_element_type=jnp.float32), 0.0)
    e1_ref[...] = jnp.dot(z1, a1_ref[...], preferred_element_type=jnp.float32)
    z2 = jnp.maximum(
        jnp.dot(ea, w2_ref[...], preferred_element_type=jnp.float32), 0.0)
    e2_ref[...] = jnp.dot(z2, a2_ref[...], preferred_element_type=jnp.float32)


def _edge_pre(edge_attr, wet1, ap1, wet2, ap2):
    blk = 2560
    de = edge_attr.shape[1]
    return pl.pallas_call(
        _edge_pre_body,
        grid=(E // blk,),
        in_specs=[
            pl.BlockSpec((blk, de), lambda i: (i, 0)),
            pl.BlockSpec((de, D), lambda i: (0, 0)),
            pl.BlockSpec((D, 8), lambda i: (0, 0)),
            pl.BlockSpec((de, D), lambda i: (0, 0)),
            pl.BlockSpec((D, 8), lambda i: (0, 0)),
        ],
        out_specs=[
            pl.BlockSpec((blk, 8), lambda i: (i, 0)),
            pl.BlockSpec((blk, 8), lambda i: (i, 0)),
        ],
        out_shape=[
            jax.ShapeDtypeStruct((E, 8), jnp.float32),
            jax.ShapeDtypeStruct((E, 8), jnp.float32),
        ],
    )(edge_attr, wet1, ap1, wet2, ap2)


def _merge_rows(acc_ref, s_ref):
    agg = jnp.concatenate([acc_ref[0], acc_ref[1]], axis=-1)
    den = s_ref[...] + 1e-16
    return agg / den


def _combine_body(acc_ref, s_ref, wt_ref, am_ref, h_ref, asd_ref):
    o = _merge_rows(acc_ref, s_ref)
    o = jnp.where(o > 0.0, o, jnp.exp(o) - 1.0)
    h = jnp.dot(o, wt_ref[...], preferred_element_type=jnp.float32)
    h_ref[...] = h
    asd_ref[...] = jnp.dot(h, am_ref[...], preferred_element_type=jnp.float32)


def _combine(acc, s, wt, attm):
    blk = 1000
    return pl.pallas_call(
        _combine_body,
        grid=(N // blk,),
        in_specs=[
            pl.BlockSpec((2, blk, DH), lambda i: (0, i, 0)),
            pl.BlockSpec((blk, 1), lambda i: (i, 0)),
            pl.BlockSpec((D, D), lambda i: (0, 0)),
            pl.BlockSpec((D, D), lambda i: (0, 0)),
        ],
        out_specs=[
            pl.BlockSpec((blk, D), lambda i: (i, 0)),
            pl.BlockSpec((blk, D), lambda i: (i, 0)),
        ],
        out_shape=[
            jax.ShapeDtypeStruct((N, D), jnp.float32),
            jax.ShapeDtypeStruct((N, D), jnp.float32),
        ],
    )(acc, s, wt, attm)


def _post_body(acc_ref, s_ref, out_ref):
    out_ref[...] = _merge_rows(acc_ref, s_ref)


def _post(acc, s):
    blk = 1000
    return pl.pallas_call(
        _post_body,
        grid=(N // blk,),
        in_specs=[
            pl.BlockSpec((2, blk, DH), lambda i: (0, i, 0)),
            pl.BlockSpec((blk, 1), lambda i: (i, 0)),
        ],
        out_specs=pl.BlockSpec((blk, D), lambda i: (i, 0)),
        out_shape=jax.ShapeDtypeStruct((N, D), jnp.float32),
    )(acc, s)


def _sc_gat_body(hs_hbm, as_hbm, ad_hbm, ep_hbm, src_hbm, dst_hbm,
                 acc_out, s_out,
                 as_v, ad_v, src_v, dst_v, ep_v, w_v, rows_v,
                 acc_sh, s_sh, sem_g, sem_i0, sem_i1, sem_s):
    c = lax.axis_index("c")
    sid = lax.axis_index("s")
    sem_i = (sem_i0, sem_i1)

    pltpu.sync_copy(as_hbm, as_v)
    pltpu.sync_copy(ad_hbm, ad_v)

    zeros16 = jnp.zeros((16,), jnp.float32)

    def _zero_row(i, _):
        for k in range(DH // 16):
            rows_v[i, pl.ds(k * 16, 16)] = zeros16
        return 0

    lax.fori_loop(0, 128, _zero_row, 0)
    for i in range(8):
        w_v[pl.ds(i * 16, 16)] = zeros16

    base_n = sid * ROWS_PER_SUB
    for m in range(ROWS_PER_SUB // 128):
        pltpu.sync_copy(rows_v.at[pl.ds(0, 128)],
                        acc_sh.at[pl.ds(base_n + m * 128, 128)])
        pltpu.sync_copy(w_v.at[pl.ds(0, 128)],
                        s_sh.at[pl.ds(base_n + m * 128, 128)])
    plsc.subcore_barrier()

    def _cid(i):
        t = sid + i * NSUB
        return jnp.minimum(t, NCHUNKS - 1), t < NCHUNKS

    def _fire_idx(i, st):
        cid, _ = _cid(i)
        base = cid * CHUNK
        for j in range(NGRP):
            pltpu.async_copy(src_hbm.at[pl.ds(base + j * 128, 128)],
                             src_v.at[st, j], sem_i[st])
            pltpu.async_copy(dst_hbm.at[pl.ds(base + j * 128, 128)],
                             dst_v.at[st, j], sem_i[st])
            pltpu.async_copy(ep_hbm.at[pl.ds(base + j * 128, 128)],
                             ep_v.at[st, j], sem_i[st])

    def _wait_idx(st):
        dummy = src_hbm.at[pl.ds(0, 128)]
        for j in range(NGRP):
            pltpu.make_async_copy(dummy, src_v.at[st, j], sem_i[st]).wait()
            pltpu.make_async_copy(dummy, dst_v.at[st, j], sem_i[st]).wait()
            pltpu.make_async_copy(
                ep_hbm.at[pl.ds(0, 128)], ep_v.at[st, j], sem_i[st]).wait()

    def _compute_w(i, st):
        _, live = _cid(i)
        fac = jnp.where(live, 1.0, 0.0)
        for j in range(NGRP):
            for t in range(8):
                o = t * 16
                s16 = src_v[st, j, pl.ds(o, 16)]
                d16 = dst_v[st, j, pl.ds(o, 16)]
                e = (plsc.load_gather(as_v, [s16])
                     + plsc.load_gather(ad_v, [d16])
                     + ep_v[st, j, pl.ds(o, 16)])
                e = jnp.where(e >= 0.0, e, 0.2 * e)
                w_v[pl.ds(j * 128 + o, 16)] = jnp.exp(e) * fac

    def _scale():
        def _body(t, _):
            w16 = w_v[pl.ds(t * 16, 16)]
            r0 = t * 16
            for l in range(16):
                wi = w16[l]
                for k in range(DH // 16):
                    rows_v[r0 + l, pl.ds(k * 16, 16)] = (
                        rows_v[r0 + l, pl.ds(k * 16, 16)] * wi)
            return 0
        lax.fori_loop(0, CHUNK // 16, _body, 0)

    def _slot(i, st):
        _wait_idx(st)
        gcps = [pltpu.async_copy(hs_hbm.at[c].at[src_v.at[st, j]],
                                 rows_v.at[pl.ds(j * 128, 128)], sem_g)
                for j in range(NGRP)]
        _fire_idx(i + 1, 1 - st)
        _compute_w(i, st)
        for cp in gcps:
            cp.wait()
        _scale()
        scps = [pltpu.async_copy(rows_v.at[pl.ds(j * 128, 128)],
                                 acc_sh.at[dst_v.at[st, j]], sem_s,
                                 add=True)
                for j in range(NGRP)]
        for cp in scps:
            cp.wait()

        @pl.when(c == 0)
        def _():
            wcps = [pltpu.async_copy(w_v.at[pl.ds(j * 128, 128)],
                                     s_sh.at[dst_v.at[st, j]], sem_s,
                                     add=True)
                    for j in range(NGRP)]
            for cp in wcps:
                cp.wait()

    _fire_idx(jnp.int32(0), 0)

    def _bdy(k, _):
        _slot(k * 2, 0)
        _slot(k * 2 + 1, 1)
        return 0

    lax.fori_loop(0, NSLOT // 2, _bdy, 0)
    _wait_idx(0)
    plsc.subcore_barrier()

    for m in range(ROWS_PER_SUB // 128):
        r0 = base_n + m * 128
        pltpu.sync_copy(acc_sh.at[pl.ds(r0, 128)],
                        acc_out.at[c, pl.ds(r0, 128)])

    @pl.when(c == 0)
    def _():
        for m in range(ROWS_PER_SUB // 128):
            r0 = base_n + m * 128
            pltpu.sync_copy(s_sh.at[pl.ds(r0, 128)],
                            s_out.at[pl.ds(r0, 128)])


def _gat_layer(hs, a_s, a_d, ep, src, dst):
    mesh = plsc.VectorSubcoreMesh(core_axis_name="c", subcore_axis_name="s",
                                  num_cores=2, num_subcores=NSUB)
    f = pl.kernel(
        _sc_gat_body,
        out_type=[
            jax.ShapeDtypeStruct((2, NP, DH), jnp.float32),
            jax.ShapeDtypeStruct((NP,), jnp.float32),
        ],
        mesh=mesh,
        compiler_params=pltpu.CompilerParams(needs_layout_passes=False,
                                             use_tc_tiling_on_sc=False),
        scratch_types=[
            pltpu.VMEM((N,), jnp.float32),
            pltpu.VMEM((N,), jnp.float32),
            pltpu.VMEM((2, NGRP, 128), jnp.int32),
            pltpu.VMEM((2, NGRP, 128), jnp.int32),
            pltpu.VMEM((2, NGRP, 128), jnp.float32),
            pltpu.VMEM((CHUNK,), jnp.float32),
            pltpu.VMEM((CHUNK, DH), jnp.float32),
            pltpu.VMEM_SHARED((NP, DH), jnp.float32),
            pltpu.VMEM_SHARED((NP,), jnp.float32),
            pltpu.SemaphoreType.DMA,
            pltpu.SemaphoreType.DMA,
            pltpu.SemaphoreType.DMA,
            pltpu.SemaphoreType.DMA,
        ],
    )
    return f(hs, a_s, a_d, ep, src, dst)


def _att_mats(att):
    a = att[0]
    attm = jnp.zeros((D, D), jnp.float32)
    attm = attm.at[:, 0].set(a[:D]).at[:, 1].set(a[D:2 * D])
    attp = jnp.zeros((D, 8), jnp.float32).at[:, 0].set(a[2 * D:])
    return attm, attp


def _split_cols(h):
    return jnp.stack([h[:, :DH], h[:, DH:]])


def kernel(x, edge_index, edge_attr, W1, We1, att1, W2, We2, att2):
    src = edge_index[0]
    dst = edge_index[1]
    attm1, attp1 = _att_mats(att1)
    attm2, attp2 = _att_mats(att2)

    h1, asd1 = _node_pre(x, W1.T, attm1)
    ep1_8, ep2_8 = _edge_pre(edge_attr, We1.T, attp1, We2.T, attp2)
    ep1 = ep1_8[:, 0]
    ep2 = ep2_8[:, 0]

    acc1, s1 = _gat_layer(_split_cols(h1), asd1[:, 0], asd1[:, 1],
                          ep1, src, dst)
    h2, asd2 = _combine(acc1, s1.reshape(NP, 1), W2.T, attm2)
    acc2, s2 = _gat_layer(_split_cols(h2), asd2[:, 0], asd2[:, 1],
                          ep2, src, dst)
    return _post(acc2, s2.reshape(NP, 1))


# speedup vs baseline: 1.3304x; 1.3294x over previous
import jax
import jax.numpy as jnp
from jax import lax
from jax.experimental import pallas as pl
from jax.experimental.pallas import tpu as pltpu
from jax.experimental.pallas import tpu_sc as plsc

N = 10000
E = 320000
D = 128
DH = D // 2
NP = 10240
ROWS_PER_SUB = NP // 16
CHUNK = 640
NGRP = CHUNK // 128
NCHUNKS = E // CHUNK
NSUB = 16
NSLOT = ((-(-NCHUNKS // NSUB) + 1) // 2) * 2


def _node_pre_body(x_ref, wt_ref, am_ref, h_ref, asd_ref):
    h = jnp.dot(x_ref[...], wt_ref[...], preferred_element_type=jnp.float32)
    h_ref[...] = h
    asd_ref[...] = jnp.dot(h, am_ref[...], preferred_element_type=jnp.float32)


def _node_pre(x, wt, attm):
    blk = 1000
    return pl.pallas_call(
        _node_pre_body,
        grid=(N // blk,),
        in_specs=[
            pl.BlockSpec((blk, D), lambda i: (i, 0)),
            pl.BlockSpec((D, D), lambda i: (0, 0)),
            pl.BlockSpec((D, D), lambda i: (0, 0)),
        ],
        out_specs=[
            pl.BlockSpec((blk, D), lambda i: (i, 0)),
            pl.BlockSpec((blk, D), lambda i: (i, 0)),
        ],
        out_shape=[
            jax.ShapeDtypeStruct((N, D), jnp.float32),
            jax.ShapeDtypeStruct((N, D), jnp.float32),
        ],
    )(x, wt, attm)


def _edge_pre_body(ea_ref, w1_ref, a1_ref, w2_ref, a2_ref, e1_ref, e2_ref):
    ea = ea_ref[...]
    z1 = jnp.maximum(
        jnp.dot(ea, w1_ref[...], preferred_element_type=jnp.float32), 0.0)
    e1_ref[...] = jnp.dot(z1, a1_ref[...], preferred_element_type=jnp.float32)
    z2 = jnp.maximum(
        jnp.dot(ea, w2_ref[...], preferred_element_type=jnp.float32), 0.0)
    e2_ref[...] = jnp.dot(z2, a2_ref[...], preferred_element_type=jnp.float32)


def _edge_pre(edge_attr, wet1, ap1, wet2, ap2):
    blk = 2560
    de = edge_attr.shape[1]
    return pl.pallas_call(
        _edge_pre_body,
        grid=(E // blk,),
        in_specs=[
            pl.BlockSpec((blk, de), lambda i: (i, 0)),
            pl.BlockSpec((de, D), lambda i: (0, 0)),
            pl.BlockSpec((D, 8), lambda i: (0, 0)),
            pl.BlockSpec((de, D), lambda i: (0, 0)),
            pl.BlockSpec((D, 8), lambda i: (0, 0)),
        ],
        out_specs=[
            pl.BlockSpec((blk, 8), lambda i: (i, 0)),
            pl.BlockSpec((blk, 8), lambda i: (i, 0)),
        ],
        out_shape=[
            jax.ShapeDtypeStruct((E, 8), jnp.float32),
            jax.ShapeDtypeStruct((E, 8), jnp.float32),
        ],
    )(edge_attr, wet1, ap1, wet2, ap2)


def _merge_rows(acc_ref, s_ref):
    agg = jnp.concatenate([acc_ref[0], acc_ref[1]], axis=-1)
    den = s_ref[...] + 1e-16
    return agg / den


def _combine_body(acc_ref, s_ref, wt_ref, am_ref, h_ref, asd_ref):
    o = _merge_rows(acc_ref, s_ref)
    o = jnp.where(o > 0.0, o, jnp.exp(o) - 1.0)
    h = jnp.dot(o, wt_ref[...], preferred_element_type=jnp.float32)
    h_ref[...] = h
    asd_ref[...] = jnp.dot(h, am_ref[...], preferred_element_type=jnp.float32)


def _combine(acc, s, wt, attm):
    blk = 1000
    return pl.pallas_call(
        _combine_body,
        grid=(N // blk,),
        in_specs=[
            pl.BlockSpec((2, blk, DH), lambda i: (0, i, 0)),
            pl.BlockSpec((blk, 1), lambda i: (i, 0)),
            pl.BlockSpec((D, D), lambda i: (0, 0)),
            pl.BlockSpec((D, D), lambda i: (0, 0)),
        ],
        out_specs=[
            pl.BlockSpec((blk, D), lambda i: (i, 0)),
            pl.BlockSpec((blk, D), lambda i: (i, 0)),
        ],
        out_shape=[
            jax.ShapeDtypeStruct((N, D), jnp.float32),
            jax.ShapeDtypeStruct((N, D), jnp.float32),
        ],
    )(acc, s, wt, attm)


def _post_body(acc_ref, s_ref, out_ref):
    out_ref[...] = _merge_rows(acc_ref, s_ref)


def _post(acc, s):
    blk = 1000
    return pl.pallas_call(
        _post_body,
        grid=(N // blk,),
        in_specs=[
            pl.BlockSpec((2, blk, DH), lambda i: (0, i, 0)),
            pl.BlockSpec((blk, 1), lambda i: (i, 0)),
        ],
        out_specs=pl.BlockSpec((blk, D), lambda i: (i, 0)),
        out_shape=jax.ShapeDtypeStruct((N, D), jnp.float32),
    )(acc, s)


def _sc_gat_body(hs_hbm, as_hbm, ad_hbm, ep_hbm, src_hbm, dst_hbm,
                 acc_out, s_out,
                 as_v, ad_v, src_v, dst_v, ep_v, w_v, rows_v,
                 acc_sh, s_sh, sem_g, sem_i0, sem_s):
    c = lax.axis_index("c")
    sid = lax.axis_index("s")

    pltpu.sync_copy(as_hbm, as_v)
    pltpu.sync_copy(ad_hbm, ad_v)

    zeros16 = jnp.zeros((16,), jnp.float32)

    def _zero_row(i, _):
        for k in range(DH // 16):
            rows_v[i, pl.ds(k * 16, 16)] = zeros16
        return 0

    lax.fori_loop(0, 128, _zero_row, 0)
    for i in range(8):
        w_v[0, pl.ds(i * 16, 16)] = zeros16

    base_n = sid * ROWS_PER_SUB
    for m in range(ROWS_PER_SUB // 128):
        pltpu.sync_copy(rows_v.at[pl.ds(0, 128)],
                        acc_sh.at[pl.ds(base_n + m * 128, 128)])
        pltpu.sync_copy(w_v.at[0],
                        s_sh.at[pl.ds(base_n + m * 128, 128)])
    plsc.subcore_barrier()

    n_c = jnp.where(sid < NCHUNKS - (NCHUNKS // NSUB) * NSUB,
                    NCHUNKS // NSUB + 1, NCHUNKS // NSUB)

    def _chunk(ci, _):
        base = (sid + ci * NSUB) * CHUNK
        icps = []
        for j in range(NGRP):
            icps.append(pltpu.async_copy(
                src_hbm.at[pl.ds(base + j * 128, 128)], src_v.at[j], sem_i0))
            icps.append(pltpu.async_copy(
                dst_hbm.at[pl.ds(base + j * 128, 128)], dst_v.at[j], sem_i0))
            icps.append(pltpu.async_copy(
                ep_hbm.at[pl.ds(base + j * 128, 128)], ep_v.at[j], sem_i0))
        for cp in icps:
            cp.wait()
        gcps = [pltpu.async_copy(hs_hbm.at[c].at[src_v.at[j]],
                                 rows_v.at[pl.ds(j * 128, 128)], sem_g)
                for j in range(NGRP)]
        for j in range(NGRP):
            for t in range(8):
                o = t * 16
                s16 = src_v[j, pl.ds(o, 16)]
                d16 = dst_v[j, pl.ds(o, 16)]
                e = (plsc.load_gather(as_v, [s16])
                     + plsc.load_gather(ad_v, [d16])
                     + ep_v[j, pl.ds(o, 16)])
                e = jnp.where(e >= 0.0, e, 0.2 * e)
                w_v[j, pl.ds(o, 16)] = jnp.exp(e)
        for cp in gcps:
            cp.wait()
        scps = []
        for j in range(NGRP):
            def _scale(i, _, j=j):
                w16 = w_v[j, pl.ds(i * 16, 16)]
                r0 = j * 128 + i * 16
                for l in range(16):
                    wi = w16[l]
                    for k in range(DH // 16):
                        rows_v[r0 + l, pl.ds(k * 16, 16)] = (
                            rows_v[r0 + l, pl.ds(k * 16, 16)] * wi)
                return 0
            lax.fori_loop(0, 8, _scale, 0)
            scps.append(pltpu.async_copy(
                rows_v.at[pl.ds(j * 128, 128)], acc_sh.at[dst_v.at[j]],
                sem_s, add=True))
            scps.append(pltpu.async_copy(
                w_v.at[j], s_sh.at[dst_v.at[j]], sem_s, add=True))
        for cp in scps:
            cp.wait()
        return 0

    lax.fori_loop(0, n_c, _chunk, 0)
    plsc.subcore_barrier()

    for m in range(ROWS_PER_SUB // 128):
        r0 = base_n + m * 128
        pltpu.sync_copy(acc_sh.at[pl.ds(r0, 128)],
                        acc_out.at[c, pl.ds(r0, 128)])

    @pl.when(c == 0)
    def _():
        for m in range(ROWS_PER_SUB // 128):
            r0 = base_n + m * 128
            pltpu.sync_copy(s_sh.at[pl.ds(r0, 128)],
                            s_out.at[pl.ds(r0, 128)])


def _gat_layer(hs, a_s, a_d, ep, src, dst):
    mesh = plsc.VectorSubcoreMesh(core_axis_name="c", subcore_axis_name="s",
                                  num_cores=2, num_subcores=NSUB)
    f = pl.kernel(
        _sc_gat_body,
        out_type=[
            jax.ShapeDtypeStruct((2, NP, DH), jnp.float32),
            jax.ShapeDtypeStruct((NP,), jnp.float32),
        ],
        mesh=mesh,
        compiler_params=pltpu.CompilerParams(needs_layout_passes=False,
                                             use_tc_tiling_on_sc=False),
        scratch_types=[
            pltpu.VMEM((N,), jnp.float32),
            pltpu.VMEM((N,), jnp.float32),
            pltpu.VMEM((NGRP, 128), jnp.int32),
            pltpu.VMEM((NGRP, 128), jnp.int32),
            pltpu.VMEM((NGRP, 128), jnp.float32),
            pltpu.VMEM((NGRP, 128), jnp.float32),
            pltpu.VMEM((CHUNK, DH), jnp.float32),
            pltpu.VMEM_SHARED((NP, DH), jnp.float32),
            pltpu.VMEM_SHARED((NP,), jnp.float32),
            pltpu.SemaphoreType.DMA,
            pltpu.SemaphoreType.DMA,
            pltpu.SemaphoreType.DMA,
        ],
    )
    return f(hs, a_s, a_d, ep, src, dst)


def _att_mats(att):
    a = att[0]
    attm = jnp.zeros((D, D), jnp.float32)
    attm = attm.at[:, 0].set(a[:D]).at[:, 1].set(a[D:2 * D])
    attp = jnp.zeros((D, 8), jnp.float32).at[:, 0].set(a[2 * D:])
    return attm, attp


def _split_cols(h):
    return jnp.stack([h[:, :DH], h[:, DH:]])


def kernel(x, edge_index, edge_attr, W1, We1, att1, W2, We2, att2):
    src = edge_index[0]
    dst = edge_index[1]
    attm1, attp1 = _att_mats(att1)
    attm2, attp2 = _att_mats(att2)

    h1, asd1 = _node_pre(x, W1.T, attm1)
    ep1_8, ep2_8 = _edge_pre(edge_attr, We1.T, attp1, We2.T, attp2)
    ep1 = ep1_8[:, 0]
    ep2 = ep2_8[:, 0]

    acc1, s1 = _gat_layer(_split_cols(h1), asd1[:, 0], asd1[:, 1],
                          ep1, src, dst)
    h2, asd2 = _combine(acc1, s1.reshape(NP, 1), W2.T, attm2)
    acc2, s2 = _gat_layer(_split_cols(h2), asd2[:, 0], asd2[:, 1],
                          ep2, src, dst)
    return _post(acc2, s2.reshape(NP, 1))


# speedup vs baseline: 1.3323x; 1.0014x over previous
import jax
import jax.numpy as jnp
from jax import lax
from jax.experimental import pallas as pl
from jax.experimental.pallas import tpu as pltpu
from jax.experimental.pallas import tpu_sc as plsc

N = 10000
E = 320000
D = 128
DH = D // 2
NP = 10240
ROWS_PER_SUB = NP // 16
CHUNK = 640
NGRP = CHUNK // 128
NCHUNKS = E // CHUNK
NSUB = 16
NSLOT = ((-(-NCHUNKS // NSUB) + 1) // 2) * 2


def _node_pre_body(x_ref, wt_ref, am_ref, h_ref, asd_ref):
    h = jnp.dot(x_ref[...], wt_ref[...], preferred_element_type=jnp.float32)
    h_ref[...] = h
    asd_ref[...] = jnp.dot(h, am_ref[...], preferred_element_type=jnp.float32)


def _node_pre(x, wt, attm):
    blk = 1000
    return pl.pallas_call(
        _node_pre_body,
        grid=(N // blk,),
        in_specs=[
            pl.BlockSpec((blk, D), lambda i: (i, 0)),
            pl.BlockSpec((D, D), lambda i: (0, 0)),
            pl.BlockSpec((D, D), lambda i: (0, 0)),
        ],
        out_specs=[
            pl.BlockSpec((blk, D), lambda i: (i, 0)),
            pl.BlockSpec((blk, D), lambda i: (i, 0)),
        ],
        out_shape=[
            jax.ShapeDtypeStruct((N, D), jnp.float32),
            jax.ShapeDtypeStruct((N, D), jnp.float32),
        ],
    )(x, wt, attm)


def _edge_pre_body(ea_ref, w1_ref, a1_ref, w2_ref, a2_ref, e1_ref, e2_ref):
    ea = ea_ref[...]
    z1 = jnp.maximum(
        jnp.dot(ea, w1_ref[...], preferred_element_type=jnp.float32), 0.0)
    e1_ref[...] = jnp.dot(z1, a1_ref[...], preferred_element_type=jnp.float32)
    z2 = jnp.maximum(
        jnp.dot(ea, w2_ref[...], preferred_element_type=jnp.float32), 0.0)
    e2_ref[...] = jnp.dot(z2, a2_ref[...], preferred_element_type=jnp.float32)


def _edge_pre(edge_attr, wet1, ap1, wet2, ap2):
    blk = 2560
    de = edge_attr.shape[1]
    return pl.pallas_call(
        _edge_pre_body,
        grid=(E // blk,),
        in_specs=[
            pl.BlockSpec((blk, de), lambda i: (i, 0)),
            pl.BlockSpec((de, D), lambda i: (0, 0)),
            pl.BlockSpec((D, 8), lambda i: (0, 0)),
            pl.BlockSpec((de, D), lambda i: (0, 0)),
            pl.BlockSpec((D, 8), lambda i: (0, 0)),
        ],
        out_specs=[
            pl.BlockSpec((blk, 8), lambda i: (i, 0)),
            pl.BlockSpec((blk, 8), lambda i: (i, 0)),
        ],
        out_shape=[
            jax.ShapeDtypeStruct((E, 8), jnp.float32),
            jax.ShapeDtypeStruct((E, 8), jnp.float32),
        ],
    )(edge_attr, wet1, ap1, wet2, ap2)


def _merge_rows(acc_ref, s_ref):
    agg = jnp.concatenate([acc_ref[0], acc_ref[1]], axis=-1)
    den = s_ref[...] + 1e-16
    return agg / den


def _combine_body(acc_ref, s_ref, wt_ref, am_ref, h_ref, asd_ref):
    o = _merge_rows(acc_ref, s_ref)
    o = jnp.where(o > 0.0, o, jnp.exp(o) - 1.0)
    h = jnp.dot(o, wt_ref[...], preferred_element_type=jnp.float32)
    h_ref[...] = h
    asd_ref[...] = jnp.dot(h, am_ref[...], preferred_element_type=jnp.float32)


def _combine(acc, s, wt, attm):
    blk = 1000
    return pl.pallas_call(
        _combine_body,
        grid=(N // blk,),
        in_specs=[
            pl.BlockSpec((2, blk, DH), lambda i: (0, i, 0)),
            pl.BlockSpec((blk, 1), lambda i: (i, 0)),
            pl.BlockSpec((D, D), lambda i: (0, 0)),
            pl.BlockSpec((D, D), lambda i: (0, 0)),
        ],
        out_specs=[
            pl.BlockSpec((blk, D), lambda i: (i, 0)),
            pl.BlockSpec((blk, D), lambda i: (i, 0)),
        ],
        out_shape=[
            jax.ShapeDtypeStruct((N, D), jnp.float32),
            jax.ShapeDtypeStruct((N, D), jnp.float32),
        ],
    )(acc, s, wt, attm)


def _post_body(acc_ref, s_ref, out_ref):
    out_ref[...] = _merge_rows(acc_ref, s_ref)


def _post(acc, s):
    blk = 1000
    return pl.pallas_call(
        _post_body,
        grid=(N // blk,),
        in_specs=[
            pl.BlockSpec((2, blk, DH), lambda i: (0, i, 0)),
            pl.BlockSpec((blk, 1), lambda i: (i, 0)),
        ],
        out_specs=pl.BlockSpec((blk, D), lambda i: (i, 0)),
        out_shape=jax.ShapeDtypeStruct((N, D), jnp.float32),
    )(acc, s)


def _sc_gat_body(hs_hbm, as_hbm, ad_hbm, ep_hbm, src_hbm, dst2_hbm,
                 acc_out, s_out,
                 as_v, ad_v, src_v, dst_v, ep_v, w_v, rows_v,
                 acc_sh, s_sh, sem_g, sem_i0, sem_s):
    c = lax.axis_index("c")
    sid = lax.axis_index("s")

    pltpu.sync_copy(as_hbm, as_v)
    pltpu.sync_copy(ad_hbm, ad_v)

    zeros16 = jnp.zeros((16,), jnp.float32)

    def _zero_row(i, _):
        for k in range(DH // 16):
            rows_v[i, pl.ds(k * 16, 16)] = zeros16
        return 0

    lax.fori_loop(0, 128, _zero_row, 0)
    for i in range(8):
        w_v[0, pl.ds(i * 16, 16)] = zeros16

    base_n = sid * ROWS_PER_SUB
    for m in range(ROWS_PER_SUB // 128):
        pltpu.sync_copy(rows_v.at[pl.ds(0, 128)],
                        acc_sh.at[pl.ds(base_n + m * 128, 128)])
        pltpu.sync_copy(w_v.at[0],
                        s_sh.at[pl.ds(base_n + m * 128, 128)])
    plsc.subcore_barrier()

    n_c = jnp.where(sid < NCHUNKS - (NCHUNKS // NSUB) * NSUB,
                    NCHUNKS // NSUB + 1, NCHUNKS // NSUB)

    def _chunk(ci, _):
        base = (sid + ci * NSUB) * CHUNK
        icps = [
            pltpu.async_copy(src_hbm.at[pl.ds(base, CHUNK)], src_v, sem_i0),
            pltpu.async_copy(
                dst2_hbm.at[pl.ds((sid + ci * NSUB) * NGRP, NGRP)],
                dst_v, sem_i0),
            pltpu.async_copy(ep_hbm.at[pl.ds(base, CHUNK)], ep_v, sem_i0),
        ]
        for cp in icps:
            cp.wait()
        gcps = [pltpu.async_copy(
                    hs_hbm.at[c].at[src_v.at[pl.ds(j * 128, 128)]],
                    rows_v.at[pl.ds(j * 128, 128)], sem_g)
                for j in range(NGRP)]
        for j in range(NGRP):
            for t in range(8):
                o = t * 16
                s16 = src_v[pl.ds(j * 128 + o, 16)]
                d16 = dst_v[j, pl.ds(o, 16)]
                e = (plsc.load_gather(as_v, [s16])
                     + plsc.load_gather(ad_v, [d16])
                     + ep_v[pl.ds(j * 128 + o, 16)])
                e = jnp.where(e >= 0.0, e, 0.2 * e)
                w_v[j, pl.ds(o, 16)] = jnp.exp(e)
        for cp in gcps:
            cp.wait()
        scps = []
        for j in range(NGRP):
            def _scale(i, _, j=j):
                w16 = w_v[j, pl.ds(i * 16, 16)]
                r0 = j * 128 + i * 16
                for l in range(16):
                    wi = w16[l]
                    for k in range(DH // 16):
                        rows_v[r0 + l, pl.ds(k * 16, 16)] = (
                            rows_v[r0 + l, pl.ds(k * 16, 16)] * wi)
                return 0
            lax.fori_loop(0, 8, _scale, 0)
            scps.append(pltpu.async_copy(
                rows_v.at[pl.ds(j * 128, 128)], acc_sh.at[dst_v.at[j]],
                sem_s, add=True))
            scps.append(pltpu.async_copy(
                w_v.at[j], s_sh.at[dst_v.at[j]], sem_s, add=True))
        for cp in scps:
            cp.wait()
        return 0

    lax.fori_loop(0, n_c, _chunk, 0)
    plsc.subcore_barrier()

    for m in range(ROWS_PER_SUB // 128):
        r0 = base_n + m * 128
        pltpu.sync_copy(acc_sh.at[pl.ds(r0, 128)],
                        acc_out.at[c, pl.ds(r0, 128)])

    @pl.when(c == 0)
    def _():
        for m in range(ROWS_PER_SUB // 128):
            r0 = base_n + m * 128
            pltpu.sync_copy(s_sh.at[pl.ds(r0, 128)],
                            s_out.at[pl.ds(r0, 128)])


def _gat_layer(hs, a_s, a_d, ep, src, dst):
    mesh = plsc.VectorSubcoreMesh(core_axis_name="c", subcore_axis_name="s",
                                  num_cores=2, num_subcores=NSUB)
    f = pl.kernel(
        _sc_gat_body,
        out_type=[
            jax.ShapeDtypeStruct((2, NP, DH), jnp.float32),
            jax.ShapeDtypeStruct((NP,), jnp.float32),
        ],
        mesh=mesh,
        compiler_params=pltpu.CompilerParams(needs_layout_passes=False,
                                             use_tc_tiling_on_sc=False),
        scratch_types=[
            pltpu.VMEM((N,), jnp.float32),
            pltpu.VMEM((N,), jnp.float32),
            pltpu.VMEM((CHUNK,), jnp.int32),
            pltpu.VMEM((NGRP, 128), jnp.int32),
            pltpu.VMEM((CHUNK,), jnp.float32),
            pltpu.VMEM((NGRP, 128), jnp.float32),
            pltpu.VMEM((CHUNK, DH), jnp.float32),
            pltpu.VMEM_SHARED((NP, DH), jnp.float32),
            pltpu.VMEM_SHARED((NP,), jnp.float32),
            pltpu.SemaphoreType.DMA,
            pltpu.SemaphoreType.DMA,
            pltpu.SemaphoreType.DMA,
        ],
    )
    return f(hs, a_s, a_d, ep, src, dst)


def _att_mats(att):
    a = att[0]
    attm = jnp.zeros((D, D), jnp.float32)
    attm = attm.at[:, 0].set(a[:D]).at[:, 1].set(a[D:2 * D])
    attp = jnp.zeros((D, 8), jnp.float32).at[:, 0].set(a[2 * D:])
    return attm, attp


def _split_cols(h):
    return jnp.stack([h[:, :DH], h[:, DH:]])


def kernel(x, edge_index, edge_attr, W1, We1, att1, W2, We2, att2):
    src = edge_index[0]
    dst = edge_index[1].reshape(E // 128, 128)
    attm1, attp1 = _att_mats(att1)
    attm2, attp2 = _att_mats(att2)

    h1, asd1 = _node_pre(x, W1.T, attm1)
    ep1_8, ep2_8 = _edge_pre(edge_attr, We1.T, attp1, We2.T, attp2)
    ep1 = ep1_8[:, 0]
    ep2 = ep2_8[:, 0]

    acc1, s1 = _gat_layer(_split_cols(h1), asd1[:, 0], asd1[:, 1],
                          ep1, src, dst)
    h2, asd2 = _combine(acc1, s1.reshape(NP, 1), W2.T, attm2)
    acc2, s2 = _gat_layer(_split_cols(h2), asd2[:, 0], asd2[:, 1],
                          ep2, src, dst)
    return _post(acc2, s2.reshape(NP, 1))
